# baseline (speedup 1.0000x reference)
import functools

import jax
import jax.numpy as jnp
from jax import lax
from jax.experimental import pallas as pl
from jax.experimental.pallas import tpu as pltpu
from jax.experimental.pallas import tpu_sc as plsc

CB = 4
NBUF = 8

_info = plsc.get_sparse_core_info()
_NW = _info.num_cores * _info.num_subcores


@functools.lru_cache(maxsize=None)
def _build(BATCH: int, HIST: int, D: int):
    b_per_w = BATCH // _NW
    n_chunks = b_per_w // CB
    idx_w = CB * HIST
    assert BATCH % (_NW * CB) == 0 and n_chunks % NBUF == 0 and idx_w % 8 == 0

    mesh = plsc.VectorSubcoreMesh(core_axis_name="c", subcore_axis_name="s")

    @functools.partial(
        pl.kernel,
        mesh=mesh,
        compiler_params=pltpu.CompilerParams(use_tc_tiling_on_sc=False),
        out_type=jax.ShapeDtypeStruct((BATCH, HIST, D), jnp.float32),
        scratch_types=[
            pltpu.VMEM((b_per_w * HIST,), jnp.int32),
            pltpu.VMEM((NBUF, idx_w, D), jnp.float32),
        ]
        + [pltpu.SemaphoreType.DMA] * (2 * NBUF),
    )
    def gather_kernel(idx_hbm, table_hbm, out_hbm, idx_v, rows_v, *sems):
        gsems, ssems = sems[:NBUF], sems[NBUF:]
        wid = lax.axis_index("s") * _info.num_cores + lax.axis_index("c")
        base = wid * b_per_w
        pltpu.sync_copy(idx_hbm.at[pl.ds(base * HIST, b_per_w * HIST)], idx_v)

        def gather(g, b):
            return pltpu.make_async_copy(
                table_hbm.at[idx_v.at[pl.ds(g * idx_w, idx_w)]],
                rows_v.at[b],
                gsems[b],
            )

        def stores(g, b):
            return [
                pltpu.make_async_copy(
                    rows_v.at[b, pl.ds(j * HIST, HIST)],
                    out_hbm.at[base + g * CB + j],
                    ssems[b],
                )
                for j in range(CB)
            ]

        for b in range(NBUF):
            gather(b, b).start()

        def ring(i, carry):
            g0 = i * NBUF
            for b in range(NBUF):
                g = g0 + b
                gather(g, b).wait()
                for s in stores(g, b):
                    s.start()
                nxt = g + NBUF - 1

                @pl.when(jnp.logical_and(g >= 1, nxt < n_chunks))
                def _():
                    bb = (b + NBUF - 1) % NBUF
                    for s in stores(g - 1, bb):
                        s.wait()
                    gather(nxt, bb).start()

            return carry

        lax.fori_loop(0, n_chunks // NBUF, ring, 0)
        for k in range(NBUF):
            g = n_chunks - NBUF + k
            for s in stores(g, g % NBUF):
                s.wait()

    return gather_kernel


def kernel(indices, table):
    bsz, hist = indices.shape
    flat = indices.reshape(bsz * hist).astype(jnp.int32)
    return _build(bsz, hist, table.shape[1])(flat, table)
